# Initial kernel scaffold; baseline (speedup 1.0000x reference)
import jax
import jax.numpy as jnp
from jax.experimental import pallas as pl


def kernel(elems, descriptors, beta, energy, rij, unique_i, unique_j, tag_i, tag_j, W1, b1, W2, b2):
    raise NotImplementedError("write your pallas kernel here")


# speedup vs baseline: 6.3678x; 6.3678x over previous
import math

import jax
import jax.numpy as jnp
import numpy as np
from jax.experimental import pallas as pl

N = 10000
K = 16
E = N * K
CUTOFF = 3.0
RMIN = 3.5
NUM_RADIAL = 5
NUM_3BODY = 12
ETA = 4.0
N_DESC = NUM_RADIAL + NUM_3BODY
HIDDEN = 128

BN = 512
NPAD = ((N + BN - 1) // BN) * BN
GRID = NPAD // BN

_MU = np.linspace(-1.0, 1.0, NUM_3BODY)
_MU_SCALE = np.exp(-ETA * _MU * _MU).astype(np.float32)
_U_COEF = np.float32(4.0 * ETA / (NUM_3BODY - 1))
_RBF_PREF = np.float32(math.sqrt(2.0 / CUTOFF))


def _pair_nn_block(r_ref, w1t_ref, b1_ref, w2_ref, b2_ref, out_ref):
    rx = r_ref[0]
    ry = r_ref[1]
    rz = r_ref[2]
    r = jnp.sqrt(rx * rx + ry * ry + rz * rz)
    rs = jnp.maximum(r, 1e-12)
    inv = 1.0 / rs

    fc = jnp.where(
        r > RMIN,
        0.5 + 0.5 * jnp.cos(np.float32(np.pi / (CUTOFF - RMIN)) * (r - RMIN)),
        1.0,
    )

    x = np.float32(np.pi / CUTOFF) * r
    s1 = jnp.sin(x)
    c1 = jnp.cos(x)
    pref = _RBF_PREF * inv * fc
    feats = [pref * s1]
    s, c = s1, c1
    for _ in range(NUM_RADIAL - 1):
        s, c = s * c1 + c * s1, c * c1 - s * s1
        feats.append(pref * s)

    fcrik = 0.5 + 0.5 * c1
    ux = rx * inv
    uy = ry * inv
    uz = rz * inv
    cos3 = (
        ux[:, None, :] * ux[None, :, :]
        + uy[:, None, :] * uy[None, :, :]
        + uz[:, None, :] * uz[None, :, :]
    )
    il = jax.lax.broadcasted_iota(jnp.int32, (K, K, 1), 0)
    ik = jax.lax.broadcasted_iota(jnp.int32, (K, K, 1), 1)
    cos3 = jnp.where(il == ik, 0.0, cos3)

    p = jnp.exp(np.float32(-ETA) * cos3 * (cos3 + 2.0)) * fcrik[:, None, :]
    u = jnp.exp(_U_COEF * cos3)
    for m in range(NUM_3BODY):
        feats.append(_MU_SCALE[m] * jnp.sum(p, axis=0))
        if m < NUM_3BODY - 1:
            p = p * u

    dmat = jnp.concatenate([f.reshape(1, K * BN) for f in feats], axis=0)

    pre = jnp.dot(w1t_ref[...], dmat, preferred_element_type=jnp.float32)
    h = jnp.tanh(pre + b1_ref[...])
    e = jnp.sum(h * w2_ref[...], axis=0, keepdims=True) + b2_ref[0, 0]
    out_ref[0] = e


def kernel(elems, descriptors, beta, energy, rij, unique_i, unique_j,
           tag_i, tag_j, W1, b1, W2, b2):
    rT = rij.reshape(N, K, 3).transpose(2, 1, 0)
    rT = jnp.pad(rT, ((0, 0), (0, 0), (0, NPAD - N)))
    out = pl.pallas_call(
        _pair_nn_block,
        grid=(GRID,),
        in_specs=[
            pl.BlockSpec((3, K, BN), lambda i: (0, 0, i)),
            pl.BlockSpec((HIDDEN, N_DESC), lambda i: (0, 0)),
            pl.BlockSpec((HIDDEN, 1), lambda i: (0, 0)),
            pl.BlockSpec((HIDDEN, 1), lambda i: (0, 0)),
            pl.BlockSpec((1, 1), lambda i: (0, 0)),
        ],
        out_specs=pl.BlockSpec((1, 1, K * BN), lambda i: (i, 0, 0)),
        out_shape=jax.ShapeDtypeStruct((GRID, 1, K * BN), jnp.float32),
    )(rT, W1.T, b1.reshape(HIDDEN, 1), W2, b2.reshape(1, 1))
    eij = out.reshape(GRID, K, BN).transpose(0, 2, 1).reshape(NPAD, K)
    return eij[:N].reshape(E, 1)
